# Initial kernel scaffold; baseline (speedup 1.0000x reference)
import jax
import jax.numpy as jnp
from jax.experimental import pallas as pl


def kernel(add_features_to_map, pillar_features, voxel_coords):
    raise NotImplementedError("write your pallas kernel here")


# speedup vs baseline: 1.2754x; 1.2754x over previous
import jax
import jax.numpy as jnp
from jax.experimental import pallas as pl

_NX, _NY, _NZ = 432, 496, 1
_B = 4
_CF = 64
_GRID = _NZ * _NX * _NY
_RTOT = _B * _GRID
_YB = 16
_SBLK = _YB * _NX
_NYB = _NY // _YB
_NBLK = _B * _NYB


def _pass_b(canvas, canvas2, idxmap, n_rows):
    idxmap3 = idxmap.reshape(_NBLK, 1, _SBLK)

    def body(cv_ref, cv2_ref, map_ref, o1_ref, o2_ref):
        occ = (map_ref[0, 0, :] > 0)
        t1 = jnp.transpose(cv_ref[...])
        o1_ref[0] = jnp.where(occ[None, :], t1, 0.0)
        t2 = jnp.transpose(cv2_ref[...])
        o2_ref[0] = jnp.where(occ[None, :], t2[:3, :], 0.0)

    out1, out2 = pl.pallas_call(
        body,
        grid=(_B, _NYB),
        in_specs=[
            pl.BlockSpec((_SBLK, _CF), lambda b, y: (b * _NYB + y, 0)),
            pl.BlockSpec((_SBLK, 8), lambda b, y: (b * _NYB + y, 0)),
            pl.BlockSpec((1, 1, _SBLK), lambda b, y: (b * _NYB + y, 0, 0)),
        ],
        out_specs=[
            pl.BlockSpec((1, _CF, _SBLK), lambda b, y: (b, 0, y)),
            pl.BlockSpec((1, 3, _SBLK), lambda b, y: (b, 0, y)),
        ],
        out_shape=[
            jax.ShapeDtypeStruct((_B, _CF, _GRID), jnp.float32),
            jax.ShapeDtypeStruct((_B, 3, _GRID), jnp.float32),
        ],
    )(canvas[:_NBLK * _SBLK], canvas2[:_NBLK * _SBLK], idxmap3)
    return out1, out2


def kernel(add_features_to_map, pillar_features, voxel_coords):
    p = pillar_features.shape[0]
    vc = voxel_coords.astype(jnp.int32)
    flat = vc[:, 0] * _GRID + vc[:, 1] + vc[:, 2] * _NX + vc[:, 3]

    ids = jnp.arange(1, p + 1, dtype=jnp.int32)
    idxmap = jnp.zeros((_RTOT,), jnp.int32).at[flat].max(ids)
    winner = jnp.maximum(idxmap - 1, 0)
    canvas = pillar_features[winner]
    addp = jnp.pad(add_features_to_map, ((0, 0), (0, 5)))
    canvas2 = addp[winner]

    out1, out2 = _pass_b(canvas, canvas2, idxmap, _RTOT)
    return (out1.reshape(_B, _CF * _NZ, _NY, _NX),
            out2.reshape(_B, 3 * _NZ, _NY, _NX))


# speedup vs baseline: 5.0630x; 3.9697x over previous
import functools

import jax
import jax.numpy as jnp
from jax import lax
from jax.experimental import pallas as pl
from jax.experimental.pallas import tpu as pltpu, tpu_sc as plsc

_NX, _NY, _NZ = 432, 496, 1
_B = 4
_CF = 64
_GRID = _NZ * _NX * _NY
_RTOT = _B * _GRID
_YB = 16
_SBLK = _YB * _NX
_NYB = _NY // _YB
_CW = 128
_CAN_ROWS = _RTOT + _SBLK
_NW = 32
_ROWN = _RTOT // _NW
_CAP = ((_ROWN + 127) // 128) * 128
_DCH = 128


def _pass_a(flat, src):
    p = flat.shape[0]
    scch = 8000
    assert p % scch == 0 and p % 16 == 0
    nch_scan = p // scch
    mesh = plsc.VectorSubcoreMesh(core_axis_name="c", subcore_axis_name="s")

    @functools.partial(
        pl.kernel,
        out_type=[
            jax.ShapeDtypeStruct((_CAN_ROWS, _CW), jnp.float32),
            jax.ShapeDtypeStruct((_RTOT,), jnp.int32),
        ],
        mesh=mesh,
        compiler_params=pltpu.CompilerParams(needs_layout_passes=False),
        scratch_types=[
            pltpu.VMEM((_ROWN,), jnp.int32),
            pltpu.VMEM((_CAP,), jnp.int32),
            pltpu.VMEM((_CAP,), jnp.int32),
            pltpu.VMEM((scch,), jnp.int32),
            pltpu.VMEM((scch,), jnp.int32),
            pltpu.VMEM((_DCH,), jnp.int32),
            pltpu.VMEM((_DCH,), jnp.int32),
            pltpu.VMEM((_DCH, _CW), jnp.float32),
            pltpu.SemaphoreType.DMA,
            pltpu.SemaphoreType.DMA,
        ],
    )
    def k(flat_hbm, src_hbm, canvas_hbm, occ_hbm,
          map_v, cells_v, winners_v, idx0, idx1, cchunk, wchunk, rowbuf,
          gsem, ssem):
        nc = 2
        wid = lax.axis_index("s") * nc + lax.axis_index("c")
        base = wid * _ROWN
        iota = lax.iota(jnp.int32, 16)
        zeros16 = jnp.zeros((16,), jnp.int32)

        def p0(i, _):
            map_v[pl.ds(i * 16, 16)] = zeros16
            return 0
        lax.fori_loop(0, _ROWN // 16, p0, 0)

        def p1(i, _):
            g = i * 16 + iota
            cells_v[pl.ds(i * 16, 16)] = _RTOT + (g & 127)
            winners_v[pl.ds(i * 16, 16)] = zeros16
            return 0
        lax.fori_loop(0, _CAP // 16, p1, 0)

        def scan_chunk(buf, c):
            def p2(i, _):
                idx = buf[pl.ds(i * 16, 16)]
                m = (idx >= base) & (idx < base + _ROWN)
                ids = c * scch + i * 16 + 1 + iota
                plsc.store_scatter(map_v, [idx - base], ids, mask=m)
                return 0
            lax.fori_loop(0, scch // 16, p2, 0)

        bufs = [idx0, idx1]
        for c in range(nch_scan):
            buf = bufs[c % 2]
            pltpu.sync_copy(flat_hbm.at[pl.ds(c * scch, scch)], buf)
            scan_chunk(buf, c)

        def p3(g, n):
            v = map_v[pl.ds(g * 16, 16)]
            m = v > 0
            plsc.store_compressed(cells_v.at[pl.ds(n, 16)],
                                  base + g * 16 + iota, mask=m)
            plsc.store_compressed(winners_v.at[pl.ds(n, 16)], v - 1, mask=m)
            return n + jnp.sum(m.astype(jnp.int32))
        n = lax.fori_loop(0, _ROWN // 16, p3, jnp.int32(0))

        def p4(j, _):
            for t in range(_DCH // 16):
                o = pl.ds(t * 16, 16)
                cchunk[o] = cells_v[pl.ds(j * _DCH + t * 16, 16)]
                wchunk[o] = winners_v[pl.ds(j * _DCH + t * 16, 16)]
            pltpu.async_copy(src_hbm.at[wchunk], rowbuf, gsem).wait()
            pltpu.async_copy(rowbuf, canvas_hbm.at[cchunk], ssem).wait()
            return 0
        lax.fori_loop(0, (n + _DCH - 1) // _DCH, p4, 0)

        pltpu.sync_copy(map_v, occ_hbm.at[pl.ds(base, _ROWN)])

    return k(flat, src)


def _pass_b(canvas, occ):
    def body(cv_ref, occ_ref, o1_ref, o2_ref):
        b = pl.program_id(0)
        rowmask = lax.broadcasted_iota(jnp.int32, (_B, 1), 0) == b
        occv = jnp.sum(jnp.where(rowmask, occ_ref[...], 0), axis=0)
        occm = (occv > 0)[None, :]
        v = cv_ref[...]
        t1 = jnp.transpose(v[:, :_CF])
        o1_ref[0] = jnp.where(occm, t1, 0.0)
        t2 = jnp.transpose(v[:, _CF:_CF + 8])
        o2_ref[0] = jnp.where(occm, t2[:3, :], 0.0)

    return pl.pallas_call(
        body,
        grid=(_B, _NYB),
        in_specs=[
            pl.BlockSpec((_SBLK, _CW), lambda b, y: (b * _NYB + y, 0)),
            pl.BlockSpec((_B, _SBLK), lambda b, y: (0, y)),
        ],
        out_specs=[
            pl.BlockSpec((1, _CF, _SBLK), lambda b, y: (b, 0, y)),
            pl.BlockSpec((1, 3, _SBLK), lambda b, y: (b, 0, y)),
        ],
        out_shape=[
            jax.ShapeDtypeStruct((_B, _CF, _GRID), jnp.float32),
            jax.ShapeDtypeStruct((_B, 3, _GRID), jnp.float32),
        ],
    )(canvas, occ)


def kernel(add_features_to_map, pillar_features, voxel_coords):
    p = pillar_features.shape[0]
    vc = voxel_coords.astype(jnp.int32)
    flat = vc[:, 0] * _GRID + vc[:, 1] + vc[:, 2] * _NX + vc[:, 3]
    src = jnp.concatenate(
        [pillar_features, add_features_to_map,
         jnp.zeros((p, _CW - _CF - 3), jnp.float32)], axis=1)

    canvas, occ = _pass_a(flat, src)
    out1, out2 = _pass_b(canvas, occ.reshape(_B, _GRID))
    return (out1.reshape(_B, _CF * _NZ, _NY, _NX),
            out2.reshape(_B, 3 * _NZ, _NY, _NX))


# speedup vs baseline: 12.9753x; 2.5628x over previous
import functools

import jax
import jax.numpy as jnp
from jax import lax
from jax.experimental import pallas as pl
from jax.experimental.pallas import tpu as pltpu, tpu_sc as plsc

_NX, _NY, _NZ = 432, 496, 1
_B = 4
_CF = 64
_GRID = _NZ * _NX * _NY
_RTOT = _B * _GRID
_YB = 16
_SBLK = _YB * _NX
_NYB = _NY // _YB
_CW = 128
_CAN_ROWS = _RTOT + _SBLK
_NW = 32
_ROWN = _RTOT // _NW
_CAP = ((_ROWN + 127) // 128) * 128
_DCH = 128


def _pass_a(flat, src):
    p = flat.shape[0]
    scch = 8000
    assert p % scch == 0 and p % 16 == 0
    nch_scan = p // scch
    mesh = plsc.VectorSubcoreMesh(core_axis_name="c", subcore_axis_name="s")

    @functools.partial(
        pl.kernel,
        out_type=[
            jax.ShapeDtypeStruct((_CAN_ROWS, _CW), jnp.float32),
            jax.ShapeDtypeStruct((_RTOT,), jnp.int32),
        ],
        mesh=mesh,
        compiler_params=pltpu.CompilerParams(needs_layout_passes=False),
        scratch_types=[
            pltpu.VMEM((_ROWN,), jnp.int32),
            pltpu.VMEM((_CAP,), jnp.int32),
            pltpu.VMEM((_CAP,), jnp.int32),
            pltpu.VMEM((scch,), jnp.int32),
            pltpu.VMEM((scch,), jnp.int32),
            pltpu.VMEM((_DCH,), jnp.int32),
            pltpu.VMEM((_DCH,), jnp.int32),
            pltpu.VMEM((_DCH, _CW), jnp.float32),
            pltpu.SemaphoreType.DMA,
            pltpu.SemaphoreType.DMA,
        ],
    )
    def k(flat_hbm, src_hbm, canvas_hbm, occ_hbm,
          map_v, cells_v, winners_v, idx0, idx1, cchunk, wchunk, rowbuf,
          gsem, ssem):
        nc = 2
        wid = lax.axis_index("s") * nc + lax.axis_index("c")
        base = wid * _ROWN
        iota = lax.iota(jnp.int32, 16)
        zeros16 = jnp.zeros((16,), jnp.int32)

        def p0(i, _):
            map_v[pl.ds(i * 16, 16)] = zeros16
            return 0
        lax.fori_loop(0, _ROWN // 16, p0, 0)

        def p1(i, _):
            g = i * 16 + iota
            cells_v[pl.ds(i * 16, 16)] = _RTOT + (g & 127)
            winners_v[pl.ds(i * 16, 16)] = zeros16
            return 0
        lax.fori_loop(0, _CAP // 16, p1, 0)

        def scan_chunk(buf, c):
            def p2(i, _):
                idx = buf[pl.ds(i * 16, 16)]
                m = (idx >= base) & (idx < base + _ROWN)
                ids = c * scch + i * 16 + 1 + iota
                plsc.store_scatter(map_v, [idx - base], ids, mask=m)
                return 0
            lax.fori_loop(0, scch // 16, p2, 0)

        bufs = [idx0, idx1]
        for c in range(nch_scan):
            buf = bufs[c % 2]
            pltpu.sync_copy(flat_hbm.at[pl.ds(c * scch, scch)], buf)
            scan_chunk(buf, c)

        def p3(g, n):
            v = map_v[pl.ds(g * 16, 16)]
            m = v > 0
            plsc.store_compressed(cells_v.at[pl.ds(n, 16)],
                                  base + g * 16 + iota, mask=m)
            plsc.store_compressed(winners_v.at[pl.ds(n, 16)], v - 1, mask=m)
            return n + jnp.sum(m.astype(jnp.int32))
        n = lax.fori_loop(0, _ROWN // 16, p3, jnp.int32(0))

        def p4(j, _):
            for t in range(_DCH // 16):
                o = pl.ds(t * 16, 16)
                cchunk[o] = cells_v[pl.ds(j * _DCH + t * 16, 16)]
                wchunk[o] = winners_v[pl.ds(j * _DCH + t * 16, 16)]
            pltpu.async_copy(src_hbm.at[wchunk], rowbuf, gsem).wait()
            pltpu.async_copy(rowbuf, canvas_hbm.at[cchunk], ssem).wait()
            return 0
        lax.fori_loop(0, (n + _DCH - 1) // _DCH, p4, 0)

        pltpu.sync_copy(map_v, occ_hbm.at[pl.ds(base, _ROWN)])

    return k(flat, src)


def _pass_b(canvas, occ):
    def body(cv_ref, occ_ref, o1_ref, o2_ref):
        b = pl.program_id(0)
        rowmask = lax.broadcasted_iota(jnp.int32, (_B, 1), 0) == b
        occv = jnp.sum(jnp.where(rowmask, occ_ref[...], 0), axis=0)
        v = cv_ref[...]
        occm = (occv > 0)[None, :]
        t = jnp.transpose(v)
        t1 = jnp.where(occm, t[:_CF], 0.0)
        t2 = jnp.where(occm, t[_CF:_CF + 3], 0.0)
        for yy in range(_YB):
            lo, hi = yy * _NX, (yy + 1) * _NX
            o1_ref[0, :, yy, :] = t1[:, lo:hi]
            o2_ref[0, :, yy, :] = t2[:, lo:hi]

    return pl.pallas_call(
        body,
        grid=(_B, _NYB),
        in_specs=[
            pl.BlockSpec((_SBLK, _CW), lambda b, y: (b * _NYB + y, 0)),
            pl.BlockSpec((_B, _SBLK), lambda b, y: (0, y)),
        ],
        out_specs=[
            pl.BlockSpec((1, _CF, _YB, _NX), lambda b, y: (b, 0, y, 0)),
            pl.BlockSpec((1, 3, _YB, _NX), lambda b, y: (b, 0, y, 0)),
        ],
        out_shape=[
            jax.ShapeDtypeStruct((_B, _CF, _NY, _NX), jnp.float32),
            jax.ShapeDtypeStruct((_B, 3, _NY, _NX), jnp.float32),
        ],
    )(canvas, occ)


def kernel(add_features_to_map, pillar_features, voxel_coords):
    p = pillar_features.shape[0]
    vc = voxel_coords.astype(jnp.int32)
    flat = vc[:, 0] * _GRID + vc[:, 1] + vc[:, 2] * _NX + vc[:, 3]
    src = jnp.concatenate(
        [pillar_features, add_features_to_map,
         jnp.zeros((p, _CW - _CF - 3), jnp.float32)], axis=1)

    canvas, occ = _pass_a(flat, src)
    return _pass_b(canvas, occ.reshape(_B, _GRID))
